# baseline (speedup 1.0000x reference)
import jax
import jax.numpy as jnp
from jax.experimental import pallas as pl
from jax.experimental.pallas import tpu as pltpu

N = 10000
F = 128
BI = 200
GRID = N // BI


def _lin0_body(x_ref, w0t_ref, b0_ref, g_ref):
    g_ref[...] = (
        jnp.dot(x_ref[...], w0t_ref[...], preferred_element_type=jnp.float32)
        + b0_ref[...]
    )


def _spmm1_body(adj_ref, g_ref, w1t_ref, b1_ref, g2_ref):
    t = jnp.dot(adj_ref[...], g_ref[...], preferred_element_type=jnp.float32)
    h = jnp.maximum(t, 0.0)
    g2_ref[...] = (
        jnp.dot(h, w1t_ref[...], preferred_element_type=jnp.float32)
        + b1_ref[...]
    )


def _spmm2_body(adj_ref, g2_ref, out_ref):
    out_ref[...] = jnp.dot(
        adj_ref[...], g2_ref[...], preferred_element_type=jnp.float32
    )


def kernel(x, adj, W0, b0, W1, b1):
    w0t = W0.T
    w1t = W1.T
    b0r = b0.reshape(1, F)
    b1r = b1.reshape(1, F)

    g = pl.pallas_call(
        _lin0_body,
        out_shape=jax.ShapeDtypeStruct((N, F), jnp.float32),
    )(x, w0t, b0r)

    row_spec = pl.BlockSpec((BI, N), lambda i: (i, 0))
    full_feat = pl.BlockSpec((N, F), lambda i: (0, 0))
    mat_spec = pl.BlockSpec((F, F), lambda i: (0, 0))
    bias_spec = pl.BlockSpec((1, F), lambda i: (0, 0))
    out_spec = pl.BlockSpec((BI, F), lambda i: (i, 0))

    g2 = pl.pallas_call(
        _spmm1_body,
        grid=(GRID,),
        in_specs=[row_spec, full_feat, mat_spec, bias_spec],
        out_specs=out_spec,
        out_shape=jax.ShapeDtypeStruct((N, F), jnp.float32),
        compiler_params=pltpu.CompilerParams(
            dimension_semantics=("arbitrary",),
        ),
    )(adj, g, w1t, b1r)

    out = pl.pallas_call(
        _spmm2_body,
        grid=(GRID,),
        in_specs=[row_spec, full_feat],
        out_specs=out_spec,
        out_shape=jax.ShapeDtypeStruct((N, F), jnp.float32),
        compiler_params=pltpu.CompilerParams(
            dimension_semantics=("arbitrary",),
        ),
    )(adj, g2)

    return out
